# Initial kernel scaffold; baseline (speedup 1.0000x reference)
import jax
import jax.numpy as jnp
from jax import lax
from jax.experimental import pallas as pl
from jax.experimental.pallas import tpu as pltpu
from jax.experimental.pallas import tpu_sc as plsc

N_NODES = 10000
N_PAD = 10240
N_EDGES = 320000
D = 128
N_GRAPHS = 64
NC, NS = 2, 16
NW = NC * NS
EPT = N_EDGES // NW
CHUNK = 80
NSTEP = EPT // CHUNK
RPT = N_PAD // NS
BLK = 1024
NBLK = 10

_f32 = jnp.float32
_i32 = jnp.int32


def _sc_mesh():
  return plsc.VectorSubcoreMesh(
      core_axis_name="c", subcore_axis_name="s", num_cores=NC,
      num_subcores=NS)


def _deg_body(dst_hbm, ew_hbm, out_hbm, didx_v, ew_v, zer_v, deg_sh):
  cid = lax.axis_index("c")
  sid = lax.axis_index("s")
  wid = cid * NS + sid

  def _zb(i, _):
    zer_v[pl.ds(i * 16, 16)] = jnp.zeros((16,), _f32)
    return 0
  lax.fori_loop(0, RPT // 16, _zb, 0)
  pltpu.sync_copy(zer_v, deg_sh.at[pl.ds(sid * RPT, RPT)])
  plsc.subcore_barrier()

  pltpu.sync_copy(dst_hbm.at[pl.ds(wid * NSTEP, NSTEP)], didx_v)
  pltpu.sync_copy(ew_hbm.at[pl.ds(wid * NSTEP, NSTEP)], ew_v)

  def _eb(k, _):
    pltpu.sync_copy(ew_v.at[k], deg_sh.at[didx_v.at[k]], add=True)
    return 0
  lax.fori_loop(0, NSTEP, _eb, 0)
  plsc.subcore_barrier()
  pltpu.sync_copy(deg_sh.at[pl.ds(sid * RPT, RPT)],
                  out_hbm.at[cid, pl.ds(sid * RPT, RPT)])


def _deg_partials(dst2, ew2):
  return pl.kernel(
      _deg_body,
      out_type=jax.ShapeDtypeStruct((NC, N_PAD), _f32),
      mesh=_sc_mesh(),
      scratch_types=[
          pltpu.VMEM((NSTEP, CHUNK), _i32),
          pltpu.VMEM((NSTEP, CHUNK), _f32),
          pltpu.VMEM((RPT,), _f32),
          pltpu.VMEM_SHARED((N_PAD,), _f32),
      ],
  )(dst2, ew2)


def _agg_body(src_hbm, dst_hbm, ew_hbm, xs_hbm, out_hbm,
              sidx_v, didx_v, ew_v, rows_v, t_sh, sem):
  cid = lax.axis_index("c")
  sid = lax.axis_index("s")
  wid = cid * NS + sid

  def _zr(i, _):
    for dch in range(D // 16):
      rows_v[i, pl.ds(dch * 16, 16)] = jnp.zeros((16,), _f32)
    return 0
  lax.fori_loop(0, CHUNK, _zr, 0)

  def _zs(j, _):
    pltpu.sync_copy(rows_v, t_sh.at[pl.ds(sid * RPT + j * CHUNK, CHUNK)])
    return 0
  lax.fori_loop(0, RPT // CHUNK, _zs, 0)
  plsc.subcore_barrier()

  pltpu.sync_copy(src_hbm.at[pl.ds(wid * NSTEP, NSTEP)], sidx_v)
  pltpu.sync_copy(dst_hbm.at[pl.ds(wid * NSTEP, NSTEP)], didx_v)
  pltpu.sync_copy(ew_hbm.at[pl.ds(wid * NSTEP, NSTEP)], ew_v)

  def _eb(k, _):
    pltpu.async_copy(xs_hbm.at[sidx_v.at[k]], rows_v, sem).wait()

    def _sb(e, _):
      s = plsc.load_gather(ew_v.at[k], [jnp.full((16,), e, _i32)])
      for dch in range(D // 16):
        sl = pl.ds(dch * 16, 16)
        rows_v[e, sl] = rows_v[e, sl] * s
      return 0
    lax.fori_loop(0, CHUNK, _sb, 0)

    pltpu.sync_copy(rows_v, t_sh.at[didx_v.at[k]], add=True)
    return 0
  lax.fori_loop(0, NSTEP, _eb, 0)
  plsc.subcore_barrier()
  pltpu.sync_copy(t_sh.at[pl.ds(sid * RPT, RPT)],
                  out_hbm.at[cid, pl.ds(sid * RPT, RPT)])


def _agg_partials(src2, dst2, ew2, xs):
  return pl.kernel(
      _agg_body,
      out_type=jax.ShapeDtypeStruct((NC, N_PAD, D), _f32),
      mesh=_sc_mesh(),
      scratch_types=[
          pltpu.VMEM((NSTEP, CHUNK), _i32),
          pltpu.VMEM((NSTEP, CHUNK), _i32),
          pltpu.VMEM((NSTEP, CHUNK), _f32),
          pltpu.VMEM((CHUNK, D), _f32),
          pltpu.VMEM_SHARED((N_PAD, D), _f32),
          pltpu.SemaphoreType.DMA,
      ],
  )(src2, dst2, ew2, xs)


def _tc1_body(x_ref, w1_ref, b1_ref, d0_ref, d1_ref, xw_ref, xs_ref):
  deg = d0_ref[...] + d1_ref[...] + 1.0
  dinv = lax.rsqrt(deg)
  xw = jnp.dot(x_ref[...], w1_ref[...],
               preferred_element_type=_f32) + b1_ref[...]
  xw_ref[...] = xw
  xs_ref[...] = xw * dinv


def _tc1(x, w1, b1r, d0, d1):
  return pl.pallas_call(
      _tc1_body,
      grid=(NBLK,),
      in_specs=[
          pl.BlockSpec((BLK, D), lambda i: (i, 0)),
          pl.BlockSpec((D, D), lambda i: (0, 0)),
          pl.BlockSpec((1, D), lambda i: (0, 0)),
          pl.BlockSpec((BLK, 1), lambda i: (i, 0)),
          pl.BlockSpec((BLK, 1), lambda i: (i, 0)),
      ],
      out_specs=[
          pl.BlockSpec((BLK, D), lambda i: (i, 0)),
          pl.BlockSpec((BLK, D), lambda i: (i, 0)),
      ],
      out_shape=[
          jax.ShapeDtypeStruct((N_NODES, D), _f32),
          jax.ShapeDtypeStruct((N_NODES, D), _f32),
      ],
  )(x, w1, b1r, d0, d1)


def _tc2_body(tp_ref, xw_ref, d0_ref, d1_ref, b_ref, w2_ref, b2_ref,
              g_ref, h_ref, pool_ref, cnt_ref):
  i = pl.program_id(0)
  deg = d0_ref[...] + d1_ref[...] + 1.0
  dinv = lax.rsqrt(deg)
  t = tp_ref[0] + tp_ref[1]
  agg = dinv * t + (dinv * dinv) * xw_ref[...]
  h = jnp.tanh(agg)
  h_ref[...] = h

  rows = lax.broadcasted_iota(_i32, (BLK, 1), 0) + i * BLK
  valid = rows < N_NODES
  gids = lax.broadcasted_iota(_i32, (BLK, N_GRAPHS), 1)
  onehot = jnp.where((b_ref[...] == gids) & valid, 1.0, 0.0)
  h_m = jnp.where(valid, h, 0.0)
  valid_f = jnp.where(valid, 1.0, 0.0)

  @pl.when(i == 0)
  def _init():
    pool_ref[...] = jnp.zeros_like(pool_ref)
    cnt_ref[...] = jnp.zeros_like(cnt_ref)

  dn = (((0,), (0,)), ((), ()))
  pool_ref[...] += lax.dot_general(onehot, h_m, dn,
                                   preferred_element_type=_f32)
  cnt_ref[...] += lax.dot_general(onehot, valid_f, dn,
                                  preferred_element_type=_f32)

  @pl.when(i == NBLK - 1)
  def _fin():
    cnt = jnp.maximum(cnt_ref[...], 1.0)
    gm = pool_ref[...] / cnt
    g = jnp.dot(gm, w2_ref[...], preferred_element_type=_f32) + b2_ref[...]
    g_ref[...] = jnp.tanh(g)


def _tc2(tp, xw, d0, d1, batch2, w2, b2r):
  return pl.pallas_call(
      _tc2_body,
      grid=(NBLK,),
      in_specs=[
          pl.BlockSpec((NC, BLK, D), lambda i: (0, i, 0)),
          pl.BlockSpec((BLK, D), lambda i: (i, 0)),
          pl.BlockSpec((BLK, 1), lambda i: (i, 0)),
          pl.BlockSpec((BLK, 1), lambda i: (i, 0)),
          pl.BlockSpec((BLK, 1), lambda i: (i, 0)),
          pl.BlockSpec((D, D), lambda i: (0, 0)),
          pl.BlockSpec((1, D), lambda i: (0, 0)),
      ],
      out_specs=[
          pl.BlockSpec((N_GRAPHS, D), lambda i: (0, 0)),
          pl.BlockSpec((BLK, D), lambda i: (i, 0)),
      ],
      out_shape=[
          jax.ShapeDtypeStruct((N_GRAPHS, D), _f32),
          jax.ShapeDtypeStruct((N_NODES, D), _f32),
      ],
      scratch_shapes=[
          pltpu.VMEM((N_GRAPHS, D), _f32),
          pltpu.VMEM((N_GRAPHS, 1), _f32),
      ],
  )(tp, xw, d0, d1, batch2, w2, b2r)


def kernel(x, edge_index, edge_features, batch, W1, b1, W2, b2):
  src2 = edge_index[0].astype(_i32).reshape(NW * NSTEP, CHUNK)
  dst2 = edge_index[1].astype(_i32).reshape(NW * NSTEP, CHUNK)
  ew2 = edge_features.astype(_f32).reshape(NW * NSTEP, CHUNK)
  batch2 = batch.astype(_i32).reshape(N_NODES, 1)

  dega = _deg_partials(dst2, ew2)
  d0 = dega[0].reshape(N_PAD, 1)
  d1 = dega[1].reshape(N_PAD, 1)
  xw, xs = _tc1(x, W1, b1.reshape(1, D), d0, d1)
  tp = _agg_partials(src2, dst2, ew2, xs)
  graph, node = _tc2(tp, xw, d0, d1, batch2, W2, b2.reshape(1, D))
  return graph, node


# speedup vs baseline: 20.1783x; 20.1783x over previous
import jax
import jax.numpy as jnp
from jax import lax
from jax.experimental import pallas as pl
from jax.experimental.pallas import tpu as pltpu
from jax.experimental.pallas import tpu_sc as plsc

N_NODES = 10000
N_PAD = 10240
N_EDGES = 320000
D = 128
N_GRAPHS = 64
NC, NS = 2, 16
NW = NC * NS
EPT = N_EDGES // NW
CHUNK = 80
NSTEP = EPT // CHUNK
RPT = N_PAD // NS
BLK = 1024
NBLK = 10

_f32 = jnp.float32
_i32 = jnp.int32


def _sc_mesh():
  return plsc.VectorSubcoreMesh(
      core_axis_name="c", subcore_axis_name="s", num_cores=NC,
      num_subcores=NS)


def _deg_body(dst_hbm, ew_hbm, out_hbm, didx_v, ew_v, zer_v, deg_sh):
  cid = lax.axis_index("c")
  sid = lax.axis_index("s")
  wid = cid * NS + sid

  def _zb(i, _):
    zer_v[pl.ds(i * 16, 16)] = jnp.zeros((16,), _f32)
    return 0
  lax.fori_loop(0, RPT // 16, _zb, 0)
  pltpu.sync_copy(zer_v, deg_sh.at[pl.ds(sid * RPT, RPT)])
  plsc.subcore_barrier()

  pltpu.sync_copy(dst_hbm.at[pl.ds(wid * NSTEP, NSTEP)], didx_v)
  pltpu.sync_copy(ew_hbm.at[pl.ds(wid * NSTEP, NSTEP)], ew_v)

  def _eb(k, _):
    pltpu.sync_copy(ew_v.at[k], deg_sh.at[didx_v.at[k]], add=True)
    return 0
  lax.fori_loop(0, NSTEP, _eb, 0)
  plsc.subcore_barrier()
  pltpu.sync_copy(deg_sh.at[pl.ds(sid * RPT, RPT)],
                  out_hbm.at[cid, pl.ds(sid * RPT, RPT)])


def _deg_partials(dst2, ew2):
  return pl.kernel(
      _deg_body,
      out_type=jax.ShapeDtypeStruct((NC, N_PAD), _f32),
      mesh=_sc_mesh(),
      compiler_params=pltpu.CompilerParams(use_tc_tiling_on_sc=False, needs_layout_passes=False),
      scratch_types=[
          pltpu.VMEM((NSTEP, CHUNK), _i32),
          pltpu.VMEM((NSTEP, CHUNK), _f32),
          pltpu.VMEM((RPT,), _f32),
          pltpu.VMEM_SHARED((N_PAD,), _f32),
      ],
  )(dst2, ew2)


def _agg_body(src_hbm, dst_hbm, ew_hbm, xs_hbm, out_hbm,
              sidx_v, didx_v, ew_v, rows_v, t_sh, sem):
  cid = lax.axis_index("c")
  sid = lax.axis_index("s")
  wid = cid * NS + sid

  def _zr(i, _):
    for dch in range(D // 16):
      rows_v[i, pl.ds(dch * 16, 16)] = jnp.zeros((16,), _f32)
    return 0
  lax.fori_loop(0, CHUNK, _zr, 0)

  def _zs(j, _):
    pltpu.sync_copy(rows_v, t_sh.at[pl.ds(sid * RPT + j * CHUNK, CHUNK)])
    return 0
  lax.fori_loop(0, RPT // CHUNK, _zs, 0)
  plsc.subcore_barrier()

  pltpu.sync_copy(src_hbm.at[pl.ds(wid * NSTEP, NSTEP)], sidx_v)
  pltpu.sync_copy(dst_hbm.at[pl.ds(wid * NSTEP, NSTEP)], didx_v)
  pltpu.sync_copy(ew_hbm.at[pl.ds(wid * NSTEP, NSTEP)], ew_v)

  def _eb(k, _):
    pltpu.async_copy(xs_hbm.at[sidx_v.at[k]], rows_v, sem).wait()

    def _sb(e, _):
      s = plsc.load_gather(ew_v.at[k], [jnp.full((16,), e, _i32)])
      for dch in range(D // 16):
        sl = pl.ds(dch * 16, 16)
        rows_v[e, sl] = rows_v[e, sl] * s
      return 0
    lax.fori_loop(0, CHUNK, _sb, 0)

    pltpu.sync_copy(rows_v, t_sh.at[didx_v.at[k]], add=True)
    return 0
  lax.fori_loop(0, NSTEP, _eb, 0)
  plsc.subcore_barrier()
  pltpu.sync_copy(t_sh.at[pl.ds(sid * RPT, RPT)],
                  out_hbm.at[cid, pl.ds(sid * RPT, RPT)])


def _agg_partials(src2, dst2, ew2, xs):
  return pl.kernel(
      _agg_body,
      out_type=jax.ShapeDtypeStruct((NC, N_PAD, D), _f32),
      mesh=_sc_mesh(),
      compiler_params=pltpu.CompilerParams(use_tc_tiling_on_sc=False, needs_layout_passes=False),
      scratch_types=[
          pltpu.VMEM((NSTEP, CHUNK), _i32),
          pltpu.VMEM((NSTEP, CHUNK), _i32),
          pltpu.VMEM((NSTEP, CHUNK), _f32),
          pltpu.VMEM((CHUNK, D), _f32),
          pltpu.VMEM_SHARED((N_PAD, D), _f32),
          pltpu.SemaphoreType.DMA,
      ],
  )(src2, dst2, ew2, xs)


def _tc1_body(x_ref, w1_ref, b1_ref, d0_ref, d1_ref, xw_ref, xs_ref):
  deg = d0_ref[...] + d1_ref[...] + 1.0
  dinv = lax.rsqrt(deg)
  xw = jnp.dot(x_ref[...], w1_ref[...],
               preferred_element_type=_f32) + b1_ref[...]
  xw_ref[...] = xw
  xs_ref[...] = xw * dinv


def _tc1(x, w1, b1r, d0, d1):
  return pl.pallas_call(
      _tc1_body,
      grid=(NBLK,),
      in_specs=[
          pl.BlockSpec((BLK, D), lambda i: (i, 0)),
          pl.BlockSpec((D, D), lambda i: (0, 0)),
          pl.BlockSpec((1, D), lambda i: (0, 0)),
          pl.BlockSpec((BLK, 1), lambda i: (i, 0)),
          pl.BlockSpec((BLK, 1), lambda i: (i, 0)),
      ],
      out_specs=[
          pl.BlockSpec((BLK, D), lambda i: (i, 0)),
          pl.BlockSpec((BLK, D), lambda i: (i, 0)),
      ],
      out_shape=[
          jax.ShapeDtypeStruct((N_NODES, D), _f32),
          jax.ShapeDtypeStruct((N_NODES, D), _f32),
      ],
  )(x, w1, b1r, d0, d1)


def _tc2_body(tp_ref, xw_ref, d0_ref, d1_ref, b_ref, w2_ref, b2_ref,
              g_ref, h_ref, pool_ref, cnt_ref):
  i = pl.program_id(0)
  deg = d0_ref[...] + d1_ref[...] + 1.0
  dinv = lax.rsqrt(deg)
  t = tp_ref[0] + tp_ref[1]
  agg = dinv * t + (dinv * dinv) * xw_ref[...]
  h = jnp.tanh(agg)
  h_ref[...] = h

  rows = lax.broadcasted_iota(_i32, (BLK, 1), 0) + i * BLK
  valid = rows < N_NODES
  gids = lax.broadcasted_iota(_i32, (BLK, N_GRAPHS), 1)
  onehot = jnp.where((b_ref[...] == gids) & valid, 1.0, 0.0)
  h_m = jnp.where(valid, h, 0.0)
  valid_f = jnp.where(valid, 1.0, 0.0)

  @pl.when(i == 0)
  def _init():
    pool_ref[...] = jnp.zeros_like(pool_ref)
    cnt_ref[...] = jnp.zeros_like(cnt_ref)

  dn = (((0,), (0,)), ((), ()))
  pool_ref[...] += lax.dot_general(onehot, h_m, dn,
                                   preferred_element_type=_f32)
  cnt_ref[...] += lax.dot_general(onehot, valid_f, dn,
                                  preferred_element_type=_f32)

  @pl.when(i == NBLK - 1)
  def _fin():
    cnt = jnp.maximum(cnt_ref[...], 1.0)
    gm = pool_ref[...] / cnt
    g = jnp.dot(gm, w2_ref[...], preferred_element_type=_f32) + b2_ref[...]
    g_ref[...] = jnp.tanh(g)


def _tc2(tp, xw, d0, d1, batch2, w2, b2r):
  return pl.pallas_call(
      _tc2_body,
      grid=(NBLK,),
      in_specs=[
          pl.BlockSpec((NC, BLK, D), lambda i: (0, i, 0)),
          pl.BlockSpec((BLK, D), lambda i: (i, 0)),
          pl.BlockSpec((BLK, 1), lambda i: (i, 0)),
          pl.BlockSpec((BLK, 1), lambda i: (i, 0)),
          pl.BlockSpec((BLK, 1), lambda i: (i, 0)),
          pl.BlockSpec((D, D), lambda i: (0, 0)),
          pl.BlockSpec((1, D), lambda i: (0, 0)),
      ],
      out_specs=[
          pl.BlockSpec((N_GRAPHS, D), lambda i: (0, 0)),
          pl.BlockSpec((BLK, D), lambda i: (i, 0)),
      ],
      out_shape=[
          jax.ShapeDtypeStruct((N_GRAPHS, D), _f32),
          jax.ShapeDtypeStruct((N_NODES, D), _f32),
      ],
      scratch_shapes=[
          pltpu.VMEM((N_GRAPHS, D), _f32),
          pltpu.VMEM((N_GRAPHS, 1), _f32),
      ],
  )(tp, xw, d0, d1, batch2, w2, b2r)


def kernel(x, edge_index, edge_features, batch, W1, b1, W2, b2):
  src2 = edge_index[0].astype(_i32).reshape(NW * NSTEP, CHUNK)
  dst2 = edge_index[1].astype(_i32).reshape(NW * NSTEP, CHUNK)
  ew2 = edge_features.astype(_f32).reshape(NW * NSTEP, CHUNK)
  batch2 = batch.astype(_i32).reshape(N_NODES, 1)

  dega = _deg_partials(dst2, ew2)
  d0 = dega[0].reshape(N_PAD, 1)
  d1 = dega[1].reshape(N_PAD, 1)
  xw, xs = _tc1(x, W1, b1.reshape(1, D), d0, d1)
  tp = _agg_partials(src2, dst2, ew2, xs)
  graph, node = _tc2(tp, xw, d0, d1, batch2, W2, b2.reshape(1, D))
  return graph, node
